# baseline (speedup 1.0000x reference)
import functools

import jax
import jax.numpy as jnp
from jax import lax
from jax.experimental import pallas as pl
from jax.experimental.pallas import tpu as pltpu
from jax.experimental.pallas import tpu_sc as plsc

_NC = 2
_NS = 16
_NW = _NC * _NS

_CH = 80

_NB = 4


def _sc_edge_aggregate(h, src, dst, n_pad):
    _, d = h.shape
    e = src.shape[0]
    per_w = e // _NW
    n_chunks = per_w // _CH
    n_groups = n_chunks // _NB
    n_tail = n_chunks - n_groups * _NB
    rows_per_tile = n_pad // _NS

    mesh = plsc.VectorSubcoreMesh(
        core_axis_name="c", subcore_axis_name="s",
        num_cores=_NC, num_subcores=_NS)

    @functools.partial(
        pl.kernel,
        out_type=jax.ShapeDtypeStruct((_NC, n_pad, d), jnp.float32),
        mesh=mesh,
        scratch_types=[
            pltpu.VMEM((2, _NB, _CH), jnp.int32),
            pltpu.VMEM((2, _NB, _CH), jnp.int32),
            [pltpu.VMEM((_CH, d), jnp.float32) for _ in range(_NB)],
            pltpu.VMEM_SHARED((n_pad, d), jnp.float32),
            pltpu.SemaphoreType.DMA,
            pltpu.SemaphoreType.DMA,
            [pltpu.SemaphoreType.DMA for _ in range(_NB)],
            pltpu.SemaphoreType.DMA,
        ],
    )
    def edge_agg(h_hbm, src_hbm, dst_hbm, out_hbm,
                 sidx, didx, rows, acc, sem_i0, sem_i1, sem_g, sem_s):
        c = lax.axis_index("c")
        s = lax.axis_index("s")
        wid = s * _NC + c
        base_w = wid * per_w

        for b in range(_NB):
            pltpu.async_copy(
                src_hbm.at[pl.ds(base_w + b * _CH, _CH)], sidx.at[0, b],
                sem_i0)
            pltpu.async_copy(
                dst_hbm.at[pl.ds(base_w + b * _CH, _CH)], didx.at[0, b],
                sem_i0)
        zv = jnp.zeros((16,), jnp.float32)

        def zrow(r, carry):
            for k8 in range(d // 16):
                rows[0][r, pl.ds(k8 * 16, 16)] = zv
            return carry

        lax.fori_loop(0, _CH, zrow, 0)
        z_full = rows_per_tile // _CH
        z_rem = rows_per_tile - z_full * _CH
        for zb in range(z_full):
            pltpu.sync_copy(
                rows[0],
                acc.at[pl.ds(s * rows_per_tile + zb * _CH, _CH)])
        if z_rem:
            pltpu.sync_copy(
                rows[0].at[pl.ds(0, z_rem)],
                acc.at[pl.ds(s * rows_per_tile + z_full * _CH, z_rem)])
        plsc.subcore_barrier()

        def group(g, carry):
            p = lax.rem(g, 2)
            @pl.when(g > 0)
            def _():
                for b in range(_NB):
                    pltpu.make_async_copy(
                        h_hbm.at[pl.ds(0, _CH)], rows[b], sem_s).wait()
            @pl.when(g + 1 < n_groups)
            def _():
                base_n = base_w + (g + 1) * _NB * _CH

                @pl.when(p == 0)
                def _():
                    for b in range(_NB):
                        pltpu.async_copy(
                            src_hbm.at[pl.ds(base_n + b * _CH, _CH)],
                            sidx.at[1, b], sem_i1)
                        pltpu.async_copy(
                            dst_hbm.at[pl.ds(base_n + b * _CH, _CH)],
                            didx.at[1, b], sem_i1)

                @pl.when(p == 1)
                def _():
                    for b in range(_NB):
                        pltpu.async_copy(
                            src_hbm.at[pl.ds(base_n + b * _CH, _CH)],
                            sidx.at[0, b], sem_i0)
                        pltpu.async_copy(
                            dst_hbm.at[pl.ds(base_n + b * _CH, _CH)],
                            didx.at[0, b], sem_i0)

            @pl.when(p == 0)
            def _():
                for b in range(_NB):
                    pltpu.make_async_copy(
                        src_hbm.at[pl.ds(0, _CH)], sidx.at[0, b],
                        sem_i0).wait()
                    pltpu.make_async_copy(
                        src_hbm.at[pl.ds(0, _CH)], didx.at[0, b],
                        sem_i0).wait()

            @pl.when(p == 1)
            def _():
                for b in range(_NB):
                    pltpu.make_async_copy(
                        src_hbm.at[pl.ds(0, _CH)], sidx.at[1, b],
                        sem_i1).wait()
                    pltpu.make_async_copy(
                        src_hbm.at[pl.ds(0, _CH)], didx.at[1, b],
                        sem_i1).wait()

            gd = []
            for b in range(_NB):
                gd.append(pltpu.async_copy(
                    h_hbm.at[sidx.at[p, b]], rows[b], sem_g[b]))
            for b in range(_NB):
                gd[b].wait()
                pltpu.async_copy(
                    rows[b], acc.at[didx.at[p, b]], sem_s, add=True)
            return carry

        lax.fori_loop(0, n_groups, group, 0)
        for b in range(_NB):
            pltpu.make_async_copy(
                h_hbm.at[pl.ds(0, _CH)], rows[b], sem_s).wait()

        for t in range(n_tail):
            base_t = base_w + (n_groups * _NB + t) * _CH
            pltpu.sync_copy(src_hbm.at[pl.ds(base_t, _CH)], sidx.at[0, 0])
            pltpu.sync_copy(dst_hbm.at[pl.ds(base_t, _CH)], didx.at[0, 0])
            pltpu.async_copy(h_hbm.at[sidx.at[0, 0]], rows[0], sem_g[0]).wait()
            pltpu.async_copy(
                rows[0], acc.at[didx.at[0, 0]], sem_s, add=True).wait()

        plsc.subcore_barrier()

        pltpu.sync_copy(
            acc.at[pl.ds(s * rows_per_tile, rows_per_tile)],
            out_hbm.at[c, pl.ds(s * rows_per_tile, rows_per_tile)])

    return edge_agg(h, src, dst)


def _linear_kernel(x_ref, w_ref, b_ref, o_ref):
    o_ref[...] = (
        jnp.dot(x_ref[...], w_ref[...], preferred_element_type=jnp.float32,
                precision=lax.Precision.HIGHEST) + b_ref[...])


def _combine_linear_kernel(p_ref, w_ref, b_ref, o_ref):
    h = jnp.maximum(p_ref[0] + p_ref[1], 0.0)
    o_ref[...] = (
        jnp.dot(h, w_ref[...], preferred_element_type=jnp.float32,
                precision=lax.Precision.HIGHEST) + b_ref[...])


def _pool_fc_kernel(p_ref, bat_ref, fw0_ref, fb0_ref, fw1_ref, fb1_ref,
                    o_ref, pooled_acc, *, g, blk, nblk):
    i = pl.program_id(0)

    @pl.when(i == 0)
    def _():
        pooled_acc[...] = jnp.zeros_like(pooled_acc)

    h = jnp.maximum(p_ref[0] + p_ref[1], 0.0)
    b = bat_ref[0]
    seg = lax.broadcasted_iota(jnp.int32, (g, blk), 0)
    onehot = (seg == b).astype(jnp.float32)
    pooled_acc[...] += jnp.dot(onehot, h, preferred_element_type=jnp.float32,
                               precision=lax.Precision.HIGHEST)

    @pl.when(i == nblk - 1)
    def _():
        pooled = pooled_acc[...]
        z = jnp.dot(pooled, fw0_ref[...], preferred_element_type=jnp.float32,
                    precision=lax.Precision.HIGHEST) + fb0_ref[...]
        z = jnp.dot(z, fw1_ref[...], preferred_element_type=jnp.float32,
                    precision=lax.Precision.HIGHEST) + fb1_ref[...]
        z = z - jnp.max(z, axis=1, keepdims=True)
        ez = jnp.exp(z)
        o_ref[...] = ez / jnp.sum(ez, axis=1, keepdims=True)


def kernel(node_attr, edge_index, batching, conv_w0, conv_b0, conv_w1,
           conv_b1, fc_w0, fc_b0, fc_w1, fc_b1):
    n, d_in = node_attr.shape
    d0 = conv_w0.shape[1]
    d1 = conv_w1.shape[1]
    f0 = fc_w0.shape[1]
    f1 = fc_w1.shape[1]
    g = 64

    src = edge_index[0]
    dst = edge_index[1]

    blk = 1000
    nblk = n // blk
    n_pad = ((n + 8 * _NS - 1) // (8 * _NS)) * (8 * _NS)

    h0 = pl.pallas_call(
        _linear_kernel,
        grid=(nblk,),
        in_specs=[
            pl.BlockSpec((blk, d_in), lambda i: (i, 0)),
            pl.BlockSpec((d_in, d0), lambda i: (0, 0)),
            pl.BlockSpec((1, d0), lambda i: (0, 0)),
        ],
        out_specs=pl.BlockSpec((blk, d0), lambda i: (i, 0)),
        out_shape=jax.ShapeDtypeStruct((n, d0), jnp.float32),
    )(node_attr, conv_w0, conv_b0.reshape(1, d0))

    part0 = _sc_edge_aggregate(h0, src, dst, n_pad)

    h1 = pl.pallas_call(
        _combine_linear_kernel,
        grid=(nblk,),
        in_specs=[
            pl.BlockSpec((_NC, blk, d0), lambda i: (0, i, 0)),
            pl.BlockSpec((d0, d1), lambda i: (0, 0)),
            pl.BlockSpec((1, d1), lambda i: (0, 0)),
        ],
        out_specs=pl.BlockSpec((blk, d1), lambda i: (i, 0)),
        out_shape=jax.ShapeDtypeStruct((n, d1), jnp.float32),
    )(part0, conv_w1, conv_b1.reshape(1, d1))

    part1 = _sc_edge_aggregate(h1, src, dst, n_pad)

    bat3 = batching.reshape(nblk, 1, blk)
    out = pl.pallas_call(
        functools.partial(_pool_fc_kernel, g=g, blk=blk, nblk=nblk),
        grid=(nblk,),
        in_specs=[
            pl.BlockSpec((_NC, blk, d1), lambda i: (0, i, 0)),
            pl.BlockSpec((1, 1, blk), lambda i: (i, 0, 0)),
            pl.BlockSpec((d1, f0), lambda i: (0, 0)),
            pl.BlockSpec((1, f0), lambda i: (0, 0)),
            pl.BlockSpec((f0, f1), lambda i: (0, 0)),
            pl.BlockSpec((1, f1), lambda i: (0, 0)),
        ],
        out_specs=pl.BlockSpec((g, f1), lambda i: (0, 0)),
        out_shape=jax.ShapeDtypeStruct((g, f1), jnp.float32),
        scratch_shapes=[pltpu.VMEM((g, d1), jnp.float32)],
    )(part1, bat3, fc_w0, fc_b0.reshape(1, f0), fc_w1, fc_b1.reshape(1, f1))

    return out


# speedup vs baseline: 1.1714x; 1.1714x over previous
import functools

import jax
import jax.numpy as jnp
from jax import lax
from jax.experimental import pallas as pl
from jax.experimental.pallas import tpu as pltpu
from jax.experimental.pallas import tpu_sc as plsc

_NC = 2
_NS = 16
_NW = _NC * _NS

_CH = 80

_NB = 4


def _sc_edge_aggregate(h, src, dst, n_pad):
    _, d = h.shape
    e = src.shape[0]
    per_w = e // _NW
    n_chunks = per_w // _CH
    n_groups = n_chunks // _NB
    n_tail = n_chunks - n_groups * _NB
    rows_per_tile = n_pad // _NS

    mesh = plsc.VectorSubcoreMesh(
        core_axis_name="c", subcore_axis_name="s",
        num_cores=_NC, num_subcores=_NS)

    @functools.partial(
        pl.kernel,
        out_type=jax.ShapeDtypeStruct((_NC, n_pad, d), jnp.float32),
        mesh=mesh,
        scratch_types=[
            pltpu.VMEM((2, _NB, _CH), jnp.int32),
            pltpu.VMEM((2, _NB, _CH), jnp.int32),
            [pltpu.VMEM((_CH, d), jnp.float32) for _ in range(_NB)],
            pltpu.VMEM_SHARED((n_pad, d), jnp.float32),
            pltpu.SemaphoreType.DMA,
            pltpu.SemaphoreType.DMA,
            pltpu.SemaphoreType.DMA,
            pltpu.SemaphoreType.DMA,
        ],
    )
    def edge_agg(h_hbm, src_hbm, dst_hbm, out_hbm,
                 sidx, didx, rows, acc, sem_i0, sem_i1, sem_g, sem_s):
        c = lax.axis_index("c")
        s = lax.axis_index("s")
        wid = s * _NC + c
        base_w = wid * per_w

        for b in range(_NB):
            pltpu.async_copy(
                src_hbm.at[pl.ds(base_w + b * _CH, _CH)], sidx.at[0, b],
                sem_i0)
            pltpu.async_copy(
                dst_hbm.at[pl.ds(base_w + b * _CH, _CH)], didx.at[0, b],
                sem_i0)
        zv = jnp.zeros((16,), jnp.float32)

        def zrow(r, carry):
            for k8 in range(d // 16):
                rows[0][r, pl.ds(k8 * 16, 16)] = zv
            return carry

        lax.fori_loop(0, _CH, zrow, 0)
        z_full = rows_per_tile // _CH
        z_rem = rows_per_tile - z_full * _CH
        for zb in range(z_full):
            pltpu.sync_copy(
                rows[0],
                acc.at[pl.ds(s * rows_per_tile + zb * _CH, _CH)])
        if z_rem:
            pltpu.sync_copy(
                rows[0].at[pl.ds(0, z_rem)],
                acc.at[pl.ds(s * rows_per_tile + z_full * _CH, z_rem)])
        plsc.subcore_barrier()

        def group(g, carry):
            p = lax.rem(g, 2)
            @pl.when(g > 0)
            def _():
                for b in range(_NB):
                    pltpu.make_async_copy(
                        h_hbm.at[pl.ds(0, _CH)], rows[b], sem_s).wait()
            @pl.when(g + 1 < n_groups)
            def _():
                base_n = base_w + (g + 1) * _NB * _CH

                @pl.when(p == 0)
                def _():
                    for b in range(_NB):
                        pltpu.async_copy(
                            src_hbm.at[pl.ds(base_n + b * _CH, _CH)],
                            sidx.at[1, b], sem_i1)
                        pltpu.async_copy(
                            dst_hbm.at[pl.ds(base_n + b * _CH, _CH)],
                            didx.at[1, b], sem_i1)

                @pl.when(p == 1)
                def _():
                    for b in range(_NB):
                        pltpu.async_copy(
                            src_hbm.at[pl.ds(base_n + b * _CH, _CH)],
                            sidx.at[0, b], sem_i0)
                        pltpu.async_copy(
                            dst_hbm.at[pl.ds(base_n + b * _CH, _CH)],
                            didx.at[0, b], sem_i0)

            @pl.when(p == 0)
            def _():
                for b in range(_NB):
                    pltpu.make_async_copy(
                        src_hbm.at[pl.ds(0, _CH)], sidx.at[0, b],
                        sem_i0).wait()
                    pltpu.make_async_copy(
                        src_hbm.at[pl.ds(0, _CH)], didx.at[0, b],
                        sem_i0).wait()

            @pl.when(p == 1)
            def _():
                for b in range(_NB):
                    pltpu.make_async_copy(
                        src_hbm.at[pl.ds(0, _CH)], sidx.at[1, b],
                        sem_i1).wait()
                    pltpu.make_async_copy(
                        src_hbm.at[pl.ds(0, _CH)], didx.at[1, b],
                        sem_i1).wait()

            gd = []
            for b in range(_NB):
                gd.append(pltpu.async_copy(
                    h_hbm.at[sidx.at[p, b]], rows[b], sem_g))
            for b in range(_NB):
                gd[b].wait()
                pltpu.async_copy(
                    rows[b], acc.at[didx.at[p, b]], sem_s, add=True)
            return carry

        lax.fori_loop(0, n_groups, group, 0)
        for b in range(_NB):
            pltpu.make_async_copy(
                h_hbm.at[pl.ds(0, _CH)], rows[b], sem_s).wait()

        for t in range(n_tail):
            base_t = base_w + (n_groups * _NB + t) * _CH
            pltpu.sync_copy(src_hbm.at[pl.ds(base_t, _CH)], sidx.at[0, 0])
            pltpu.sync_copy(dst_hbm.at[pl.ds(base_t, _CH)], didx.at[0, 0])
            pltpu.async_copy(h_hbm.at[sidx.at[0, 0]], rows[0], sem_g).wait()
            pltpu.async_copy(
                rows[0], acc.at[didx.at[0, 0]], sem_s, add=True).wait()

        plsc.subcore_barrier()

        pltpu.sync_copy(
            acc.at[pl.ds(s * rows_per_tile, rows_per_tile)],
            out_hbm.at[c, pl.ds(s * rows_per_tile, rows_per_tile)])

    return edge_agg(h, src, dst)


def _dot_bf16x3(x, w):
    xh = x.astype(jnp.bfloat16)
    wh = w.astype(jnp.bfloat16)
    xl = (x - xh.astype(jnp.float32)).astype(jnp.bfloat16)
    wl = (w - wh.astype(jnp.float32)).astype(jnp.bfloat16)
    f = jnp.float32
    return (jnp.dot(xh, wh, preferred_element_type=f)
            + jnp.dot(xh, wl, preferred_element_type=f)
            + jnp.dot(xl, wh, preferred_element_type=f))


def _linear_kernel(x_ref, w_ref, b_ref, o_ref):
    o_ref[...] = _dot_bf16x3(x_ref[...], w_ref[...]) + b_ref[...]


def _combine_linear_kernel(p_ref, w_ref, b_ref, o_ref):
    h = jnp.maximum(p_ref[0] + p_ref[1], 0.0)
    o_ref[...] = _dot_bf16x3(h, w_ref[...]) + b_ref[...]


def _pool_fc_kernel(p_ref, bat_ref, fw0_ref, fb0_ref, fw1_ref, fb1_ref,
                    o_ref, pooled_acc, *, g, blk, nblk):
    i = pl.program_id(0)

    @pl.when(i == 0)
    def _():
        pooled_acc[...] = jnp.zeros_like(pooled_acc)

    h = jnp.maximum(p_ref[0] + p_ref[1], 0.0)
    b = bat_ref[0]
    seg = lax.broadcasted_iota(jnp.int32, (g, blk), 0)
    onehot = (seg == b).astype(jnp.float32)
    pooled_acc[...] += jnp.dot(onehot, h, preferred_element_type=jnp.float32,
                               precision=lax.Precision.HIGHEST)

    @pl.when(i == nblk - 1)
    def _():
        pooled = pooled_acc[...]
        z = jnp.dot(pooled, fw0_ref[...], preferred_element_type=jnp.float32,
                    precision=lax.Precision.HIGHEST) + fb0_ref[...]
        z = jnp.dot(z, fw1_ref[...], preferred_element_type=jnp.float32,
                    precision=lax.Precision.HIGHEST) + fb1_ref[...]
        z = z - jnp.max(z, axis=1, keepdims=True)
        ez = jnp.exp(z)
        o_ref[...] = ez / jnp.sum(ez, axis=1, keepdims=True)


def kernel(node_attr, edge_index, batching, conv_w0, conv_b0, conv_w1,
           conv_b1, fc_w0, fc_b0, fc_w1, fc_b1):
    n, d_in = node_attr.shape
    d0 = conv_w0.shape[1]
    d1 = conv_w1.shape[1]
    f0 = fc_w0.shape[1]
    f1 = fc_w1.shape[1]
    g = 64

    src = edge_index[0]
    dst = edge_index[1]

    blk = 1000
    nblk = n // blk
    n_pad = ((n + 8 * _NS - 1) // (8 * _NS)) * (8 * _NS)

    h0 = pl.pallas_call(
        _linear_kernel,
        grid=(nblk,),
        in_specs=[
            pl.BlockSpec((blk, d_in), lambda i: (i, 0)),
            pl.BlockSpec((d_in, d0), lambda i: (0, 0)),
            pl.BlockSpec((1, d0), lambda i: (0, 0)),
        ],
        out_specs=pl.BlockSpec((blk, d0), lambda i: (i, 0)),
        out_shape=jax.ShapeDtypeStruct((n, d0), jnp.float32),
    )(node_attr, conv_w0, conv_b0.reshape(1, d0))

    part0 = _sc_edge_aggregate(h0, src, dst, n_pad)

    h1 = pl.pallas_call(
        _combine_linear_kernel,
        grid=(nblk,),
        in_specs=[
            pl.BlockSpec((_NC, blk, d0), lambda i: (0, i, 0)),
            pl.BlockSpec((d0, d1), lambda i: (0, 0)),
            pl.BlockSpec((1, d1), lambda i: (0, 0)),
        ],
        out_specs=pl.BlockSpec((blk, d1), lambda i: (i, 0)),
        out_shape=jax.ShapeDtypeStruct((n, d1), jnp.float32),
    )(part0, conv_w1, conv_b1.reshape(1, d1))

    part1 = _sc_edge_aggregate(h1, src, dst, n_pad)

    bat3 = batching.reshape(nblk, 1, blk)
    out = pl.pallas_call(
        functools.partial(_pool_fc_kernel, g=g, blk=blk, nblk=nblk),
        grid=(nblk,),
        in_specs=[
            pl.BlockSpec((_NC, blk, d1), lambda i: (0, i, 0)),
            pl.BlockSpec((1, 1, blk), lambda i: (i, 0, 0)),
            pl.BlockSpec((d1, f0), lambda i: (0, 0)),
            pl.BlockSpec((1, f0), lambda i: (0, 0)),
            pl.BlockSpec((f0, f1), lambda i: (0, 0)),
            pl.BlockSpec((1, f1), lambda i: (0, 0)),
        ],
        out_specs=pl.BlockSpec((g, f1), lambda i: (0, 0)),
        out_shape=jax.ShapeDtypeStruct((g, f1), jnp.float32),
        scratch_shapes=[pltpu.VMEM((g, d1), jnp.float32)],
    )(part1, bat3, fc_w0, fc_b0.reshape(1, f0), fc_w1, fc_b1.reshape(1, f1))

    return out


# speedup vs baseline: 1.1764x; 1.0043x over previous
import functools

import jax
import jax.numpy as jnp
from jax import lax
from jax.experimental import pallas as pl
from jax.experimental.pallas import tpu as pltpu
from jax.experimental.pallas import tpu_sc as plsc

_NC = 2
_NS = 16
_NW = _NC * _NS

_CH = 80

_NB = 4


def _sc_edge_aggregate(h, src, dst, n_pad):
    _, d = h.shape
    e = src.shape[0]
    per_w = e // _NW
    n_chunks = per_w // _CH
    n_groups = n_chunks // _NB
    n_tail = n_chunks - n_groups * _NB
    rows_per_tile = n_pad // _NS

    mesh = plsc.VectorSubcoreMesh(
        core_axis_name="c", subcore_axis_name="s",
        num_cores=_NC, num_subcores=_NS)

    @functools.partial(
        pl.kernel,
        out_type=jax.ShapeDtypeStruct((_NC, n_pad, d), jnp.float32),
        mesh=mesh,
        scratch_types=[
            pltpu.VMEM((2, _NB, _CH), jnp.int32),
            pltpu.VMEM((2, _NB, _CH), jnp.int32),
            [pltpu.VMEM((_CH, d), jnp.float32) for _ in range(_NB)],
            pltpu.VMEM_SHARED((n_pad, d), jnp.float32),
            pltpu.SemaphoreType.DMA,
            pltpu.SemaphoreType.DMA,
            pltpu.SemaphoreType.DMA,
            pltpu.SemaphoreType.DMA,
        ],
    )
    def edge_agg(h_hbm, src_hbm, dst_hbm, out_hbm,
                 sidx, didx, rows, acc, sem_i0, sem_i1, sem_g, sem_s):
        c = lax.axis_index("c")
        s = lax.axis_index("s")
        wid = s * _NC + c
        base_w = wid * per_w

        for b in range(_NB):
            pltpu.async_copy(
                src_hbm.at[pl.ds(base_w + b * _CH, _CH)], sidx.at[0, b],
                sem_i0)
            pltpu.async_copy(
                dst_hbm.at[pl.ds(base_w + b * _CH, _CH)], didx.at[0, b],
                sem_i0)
        zv = jnp.zeros((16,), jnp.float32)

        def zrow(r, carry):
            for k8 in range(d // 16):
                rows[0][r, pl.ds(k8 * 16, 16)] = zv
            return carry

        lax.fori_loop(0, _CH, zrow, 0)
        z_full = rows_per_tile // _CH
        z_rem = rows_per_tile - z_full * _CH
        for zb in range(z_full):
            pltpu.sync_copy(
                rows[0],
                acc.at[pl.ds(s * rows_per_tile + zb * _CH, _CH)])
        if z_rem:
            pltpu.sync_copy(
                rows[0].at[pl.ds(0, z_rem)],
                acc.at[pl.ds(s * rows_per_tile + z_full * _CH, z_rem)])
        plsc.subcore_barrier()

        def group(g, carry):
            p = lax.rem(g, 2)
            @pl.when(g > 0)
            def _():
                for b in range(_NB):
                    pltpu.make_async_copy(
                        h_hbm.at[pl.ds(0, _CH)], rows[b], sem_s).wait()
            @pl.when(g + 1 < n_groups)
            def _():
                base_n = base_w + (g + 1) * _NB * _CH

                @pl.when(p == 0)
                def _():
                    for b in range(_NB):
                        pltpu.async_copy(
                            src_hbm.at[pl.ds(base_n + b * _CH, _CH)],
                            sidx.at[1, b], sem_i1)
                        pltpu.async_copy(
                            dst_hbm.at[pl.ds(base_n + b * _CH, _CH)],
                            didx.at[1, b], sem_i1)

                @pl.when(p == 1)
                def _():
                    for b in range(_NB):
                        pltpu.async_copy(
                            src_hbm.at[pl.ds(base_n + b * _CH, _CH)],
                            sidx.at[0, b], sem_i0)
                        pltpu.async_copy(
                            dst_hbm.at[pl.ds(base_n + b * _CH, _CH)],
                            didx.at[0, b], sem_i0)

            @pl.when(p == 0)
            def _():
                for b in range(_NB):
                    pltpu.make_async_copy(
                        src_hbm.at[pl.ds(0, _CH)], sidx.at[0, b],
                        sem_i0).wait()
                    pltpu.make_async_copy(
                        src_hbm.at[pl.ds(0, _CH)], didx.at[0, b],
                        sem_i0).wait()

            @pl.when(p == 1)
            def _():
                for b in range(_NB):
                    pltpu.make_async_copy(
                        src_hbm.at[pl.ds(0, _CH)], sidx.at[1, b],
                        sem_i1).wait()
                    pltpu.make_async_copy(
                        src_hbm.at[pl.ds(0, _CH)], didx.at[1, b],
                        sem_i1).wait()

            gd = []
            for b in range(_NB):
                gd.append(pltpu.async_copy(
                    h_hbm.at[sidx.at[p, b]], rows[b], sem_g))
            for b in range(_NB):
                gd[b].wait()
                pltpu.async_copy(
                    rows[b], acc.at[didx.at[p, b]], sem_s, add=True)
            return carry

        lax.fori_loop(0, n_groups, group, 0)
        for b in range(_NB):
            pltpu.make_async_copy(
                h_hbm.at[pl.ds(0, _CH)], rows[b], sem_s).wait()

        for t in range(n_tail):
            base_t = base_w + (n_groups * _NB + t) * _CH
            pltpu.sync_copy(src_hbm.at[pl.ds(base_t, _CH)], sidx.at[0, 0])
            pltpu.sync_copy(dst_hbm.at[pl.ds(base_t, _CH)], didx.at[0, 0])
            pltpu.async_copy(h_hbm.at[sidx.at[0, 0]], rows[0], sem_g).wait()
            pltpu.async_copy(
                rows[0], acc.at[didx.at[0, 0]], sem_s, add=True).wait()

        plsc.subcore_barrier()

        pltpu.sync_copy(
            acc.at[pl.ds(s * rows_per_tile, rows_per_tile)],
            out_hbm.at[c, pl.ds(s * rows_per_tile, rows_per_tile)])

    return edge_agg(h, src, dst)


def _dot_bf16x3(x, w):
    xh = x.astype(jnp.bfloat16)
    wh = w.astype(jnp.bfloat16)
    xl = (x - xh.astype(jnp.float32)).astype(jnp.bfloat16)
    wl = (w - wh.astype(jnp.float32)).astype(jnp.bfloat16)
    f = jnp.float32
    return (jnp.dot(xh, wh, preferred_element_type=f)
            + jnp.dot(xh, wl, preferred_element_type=f)
            + jnp.dot(xl, wh, preferred_element_type=f))


def _linear_kernel(x_ref, w_ref, b_ref, o_ref):
    o_ref[...] = _dot_bf16x3(x_ref[...], w_ref[...]) + b_ref[...]


def _combine_linear_kernel(p_ref, w_ref, b_ref, o_ref):
    h = jnp.maximum(p_ref[0] + p_ref[1], 0.0)
    o_ref[...] = _dot_bf16x3(h, w_ref[...]) + b_ref[...]


def _pool_fc_kernel(p_ref, bat_ref, fw0_ref, fb0_ref, fw1_ref, fb1_ref,
                    o_ref, pooled_acc, *, g, blk, nblk):
    i = pl.program_id(0)

    @pl.when(i == 0)
    def _():
        pooled_acc[...] = jnp.zeros_like(pooled_acc)

    h = jnp.maximum(p_ref[0] + p_ref[1], 0.0)
    b = bat_ref[0]
    seg = lax.broadcasted_iota(jnp.int32, (g, blk), 0)
    onehot = (seg == b).astype(jnp.bfloat16)
    hh = h.astype(jnp.bfloat16)
    hl = (h - hh.astype(jnp.float32)).astype(jnp.bfloat16)
    pooled_acc[...] += (
        jnp.dot(onehot, hh, preferred_element_type=jnp.float32)
        + jnp.dot(onehot, hl, preferred_element_type=jnp.float32))

    @pl.when(i == nblk - 1)
    def _():
        pooled = pooled_acc[...]
        z = jnp.dot(pooled, fw0_ref[...], preferred_element_type=jnp.float32,
                    precision=lax.Precision.HIGHEST) + fb0_ref[...]
        z = jnp.dot(z, fw1_ref[...], preferred_element_type=jnp.float32,
                    precision=lax.Precision.HIGHEST) + fb1_ref[...]
        z = z - jnp.max(z, axis=1, keepdims=True)
        ez = jnp.exp(z)
        o_ref[...] = ez / jnp.sum(ez, axis=1, keepdims=True)


def kernel(node_attr, edge_index, batching, conv_w0, conv_b0, conv_w1,
           conv_b1, fc_w0, fc_b0, fc_w1, fc_b1):
    n, d_in = node_attr.shape
    d0 = conv_w0.shape[1]
    d1 = conv_w1.shape[1]
    f0 = fc_w0.shape[1]
    f1 = fc_w1.shape[1]
    g = 64

    src = edge_index[0]
    dst = edge_index[1]

    blk = 1000
    nblk = n // blk
    n_pad = ((n + 8 * _NS - 1) // (8 * _NS)) * (8 * _NS)

    h0 = pl.pallas_call(
        _linear_kernel,
        grid=(nblk,),
        in_specs=[
            pl.BlockSpec((blk, d_in), lambda i: (i, 0)),
            pl.BlockSpec((d_in, d0), lambda i: (0, 0)),
            pl.BlockSpec((1, d0), lambda i: (0, 0)),
        ],
        out_specs=pl.BlockSpec((blk, d0), lambda i: (i, 0)),
        out_shape=jax.ShapeDtypeStruct((n, d0), jnp.float32),
    )(node_attr, conv_w0, conv_b0.reshape(1, d0))

    part0 = _sc_edge_aggregate(h0, src, dst, n_pad)

    h1 = pl.pallas_call(
        _combine_linear_kernel,
        grid=(nblk,),
        in_specs=[
            pl.BlockSpec((_NC, blk, d0), lambda i: (0, i, 0)),
            pl.BlockSpec((d0, d1), lambda i: (0, 0)),
            pl.BlockSpec((1, d1), lambda i: (0, 0)),
        ],
        out_specs=pl.BlockSpec((blk, d1), lambda i: (i, 0)),
        out_shape=jax.ShapeDtypeStruct((n, d1), jnp.float32),
    )(part0, conv_w1, conv_b1.reshape(1, d1))

    part1 = _sc_edge_aggregate(h1, src, dst, n_pad)

    bat3 = batching.reshape(nblk, 1, blk)
    out = pl.pallas_call(
        functools.partial(_pool_fc_kernel, g=g, blk=blk, nblk=nblk),
        grid=(nblk,),
        in_specs=[
            pl.BlockSpec((_NC, blk, d1), lambda i: (0, i, 0)),
            pl.BlockSpec((1, 1, blk), lambda i: (i, 0, 0)),
            pl.BlockSpec((d1, f0), lambda i: (0, 0)),
            pl.BlockSpec((1, f0), lambda i: (0, 0)),
            pl.BlockSpec((f0, f1), lambda i: (0, 0)),
            pl.BlockSpec((1, f1), lambda i: (0, 0)),
        ],
        out_specs=pl.BlockSpec((g, f1), lambda i: (0, 0)),
        out_shape=jax.ShapeDtypeStruct((g, f1), jnp.float32),
        scratch_shapes=[pltpu.VMEM((g, d1), jnp.float32)],
    )(part1, bat3, fc_w0, fc_b0.reshape(1, f0), fc_w1, fc_b1.reshape(1, f1))

    return out


# speedup vs baseline: 1.2161x; 1.0338x over previous
import functools

import jax
import jax.numpy as jnp
from jax import lax
from jax.experimental import pallas as pl
from jax.experimental.pallas import tpu as pltpu
from jax.experimental.pallas import tpu_sc as plsc

_NC = 2
_NS = 16
_NW = _NC * _NS

_CH = 80

_NB = 4


def _sc_edge_aggregate(h, eidx_flat, n_pad):
    _, d = h.shape
    e = eidx_flat.shape[0] // 2
    per_w = e // _NW
    n_chunks = per_w // _CH
    n_groups = n_chunks // _NB
    n_tail = n_chunks - n_groups * _NB
    rows_per_tile = n_pad // _NS

    mesh = plsc.VectorSubcoreMesh(
        core_axis_name="c", subcore_axis_name="s",
        num_cores=_NC, num_subcores=_NS)

    @functools.partial(
        pl.kernel,
        out_type=jax.ShapeDtypeStruct((_NC, n_pad, d), jnp.float32),
        mesh=mesh,
        scratch_types=[
            pltpu.VMEM((2, _NB, _CH), jnp.int32),
            pltpu.VMEM((2, _NB, _CH), jnp.int32),
            [pltpu.VMEM((_CH, d), jnp.float32) for _ in range(_NB)],
            pltpu.VMEM_SHARED((n_pad, d), jnp.float32),
            pltpu.SemaphoreType.DMA,
            pltpu.SemaphoreType.DMA,
            pltpu.SemaphoreType.DMA,
            pltpu.SemaphoreType.DMA,
        ],
    )
    def edge_agg(h_hbm, eidx_hbm, out_hbm,
                 sidx, didx, rows, acc, sem_i0, sem_i1, sem_g, sem_s):
        c = lax.axis_index("c")
        s = lax.axis_index("s")
        wid = s * _NC + c
        base_w = wid * per_w

        for b in range(_NB):
            pltpu.async_copy(
                eidx_hbm.at[pl.ds(base_w + b * _CH, _CH)], sidx.at[0, b],
                sem_i0)
            pltpu.async_copy(
                eidx_hbm.at[pl.ds(e + base_w + b * _CH, _CH)], didx.at[0, b],
                sem_i0)
        zv = jnp.zeros((16,), jnp.float32)

        def zrow(r, carry):
            for k8 in range(d // 16):
                rows[0][r, pl.ds(k8 * 16, 16)] = zv
            return carry

        lax.fori_loop(0, _CH, zrow, 0)
        z_full = rows_per_tile // _CH
        z_rem = rows_per_tile - z_full * _CH
        for zb in range(z_full):
            pltpu.sync_copy(
                rows[0],
                acc.at[pl.ds(s * rows_per_tile + zb * _CH, _CH)])
        if z_rem:
            pltpu.sync_copy(
                rows[0].at[pl.ds(0, z_rem)],
                acc.at[pl.ds(s * rows_per_tile + z_full * _CH, z_rem)])
        plsc.subcore_barrier()

        def group(g, carry):
            p = lax.rem(g, 2)
            @pl.when(g > 0)
            def _():
                for b in range(_NB):
                    pltpu.make_async_copy(
                        h_hbm.at[pl.ds(0, _CH)], rows[b], sem_s).wait()
            @pl.when(g + 1 < n_groups)
            def _():
                base_n = base_w + (g + 1) * _NB * _CH

                @pl.when(p == 0)
                def _():
                    for b in range(_NB):
                        pltpu.async_copy(
                            eidx_hbm.at[pl.ds(base_n + b * _CH, _CH)],
                            sidx.at[1, b], sem_i1)
                        pltpu.async_copy(
                            eidx_hbm.at[pl.ds(e + base_n + b * _CH, _CH)],
                            didx.at[1, b], sem_i1)

                @pl.when(p == 1)
                def _():
                    for b in range(_NB):
                        pltpu.async_copy(
                            eidx_hbm.at[pl.ds(base_n + b * _CH, _CH)],
                            sidx.at[0, b], sem_i0)
                        pltpu.async_copy(
                            eidx_hbm.at[pl.ds(e + base_n + b * _CH, _CH)],
                            didx.at[0, b], sem_i0)

            @pl.when(p == 0)
            def _():
                for b in range(_NB):
                    pltpu.make_async_copy(
                        eidx_hbm.at[pl.ds(0, _CH)], sidx.at[0, b],
                        sem_i0).wait()
                    pltpu.make_async_copy(
                        eidx_hbm.at[pl.ds(0, _CH)], didx.at[0, b],
                        sem_i0).wait()

            @pl.when(p == 1)
            def _():
                for b in range(_NB):
                    pltpu.make_async_copy(
                        eidx_hbm.at[pl.ds(0, _CH)], sidx.at[1, b],
                        sem_i1).wait()
                    pltpu.make_async_copy(
                        eidx_hbm.at[pl.ds(0, _CH)], didx.at[1, b],
                        sem_i1).wait()

            gd = []
            for b in range(_NB):
                gd.append(pltpu.async_copy(
                    h_hbm.at[sidx.at[p, b]], rows[b], sem_g))
            for b in range(_NB):
                gd[b].wait()
                pltpu.async_copy(
                    rows[b], acc.at[didx.at[p, b]], sem_s, add=True)
            return carry

        lax.fori_loop(0, n_groups, group, 0)
        for b in range(_NB):
            pltpu.make_async_copy(
                h_hbm.at[pl.ds(0, _CH)], rows[b], sem_s).wait()

        for t in range(n_tail):
            base_t = base_w + (n_groups * _NB + t) * _CH
            pltpu.sync_copy(eidx_hbm.at[pl.ds(base_t, _CH)], sidx.at[0, 0])
            pltpu.sync_copy(eidx_hbm.at[pl.ds(e + base_t, _CH)], didx.at[0, 0])
            pltpu.async_copy(h_hbm.at[sidx.at[0, 0]], rows[0], sem_g).wait()
            pltpu.async_copy(
                rows[0], acc.at[didx.at[0, 0]], sem_s, add=True).wait()

        plsc.subcore_barrier()

        pltpu.sync_copy(
            acc.at[pl.ds(s * rows_per_tile, rows_per_tile)],
            out_hbm.at[c, pl.ds(s * rows_per_tile, rows_per_tile)])

    return edge_agg(h, eidx_flat)


def _dot_bf16x3(x, w):
    xh = x.astype(jnp.bfloat16)
    wh = w.astype(jnp.bfloat16)
    xl = (x - xh.astype(jnp.float32)).astype(jnp.bfloat16)
    wl = (w - wh.astype(jnp.float32)).astype(jnp.bfloat16)
    f = jnp.float32
    return (jnp.dot(xh, wh, preferred_element_type=f)
            + jnp.dot(xh, wl, preferred_element_type=f)
            + jnp.dot(xl, wh, preferred_element_type=f))


def _linear_kernel(x_ref, w_ref, b_ref, o_ref):
    o_ref[...] = _dot_bf16x3(x_ref[...], w_ref[...]) + b_ref[...]


def _combine_linear_kernel(p_ref, w_ref, b_ref, o_ref):
    h = jnp.maximum(p_ref[0] + p_ref[1], 0.0)
    o_ref[...] = _dot_bf16x3(h, w_ref[...]) + b_ref[...]


def _pool_fc_kernel(p_ref, bat_ref, fw0_ref, fb0_ref, fw1_ref, fb1_ref,
                    o_ref, pooled_acc, *, g, blk, nblk):
    i = pl.program_id(0)

    @pl.when(i == 0)
    def _():
        pooled_acc[...] = jnp.zeros_like(pooled_acc)

    h = jnp.maximum(p_ref[0] + p_ref[1], 0.0)
    b = bat_ref[0]
    seg = lax.broadcasted_iota(jnp.int32, (g, blk), 0)
    onehot = (seg == b).astype(jnp.bfloat16)
    hh = h.astype(jnp.bfloat16)
    hl = (h - hh.astype(jnp.float32)).astype(jnp.bfloat16)
    pooled_acc[...] += (
        jnp.dot(onehot, hh, preferred_element_type=jnp.float32)
        + jnp.dot(onehot, hl, preferred_element_type=jnp.float32))

    @pl.when(i == nblk - 1)
    def _():
        pooled = pooled_acc[...]
        z = jnp.dot(pooled, fw0_ref[...], preferred_element_type=jnp.float32,
                    precision=lax.Precision.HIGHEST) + fb0_ref[...]
        z = jnp.dot(z, fw1_ref[...], preferred_element_type=jnp.float32,
                    precision=lax.Precision.HIGHEST) + fb1_ref[...]
        z = z - jnp.max(z, axis=1, keepdims=True)
        ez = jnp.exp(z)
        o_ref[...] = ez / jnp.sum(ez, axis=1, keepdims=True)


def kernel(node_attr, edge_index, batching, conv_w0, conv_b0, conv_w1,
           conv_b1, fc_w0, fc_b0, fc_w1, fc_b1):
    n, d_in = node_attr.shape
    d0 = conv_w0.shape[1]
    d1 = conv_w1.shape[1]
    f0 = fc_w0.shape[1]
    f1 = fc_w1.shape[1]
    g = 64

    eidx_flat = edge_index.reshape(-1)

    blk = 1000
    nblk = n // blk
    n_pad = ((n + 8 * _NS - 1) // (8 * _NS)) * (8 * _NS)

    h0 = pl.pallas_call(
        _linear_kernel,
        grid=(nblk,),
        in_specs=[
            pl.BlockSpec((blk, d_in), lambda i: (i, 0)),
            pl.BlockSpec((d_in, d0), lambda i: (0, 0)),
            pl.BlockSpec((1, d0), lambda i: (0, 0)),
        ],
        out_specs=pl.BlockSpec((blk, d0), lambda i: (i, 0)),
        out_shape=jax.ShapeDtypeStruct((n, d0), jnp.float32),
    )(node_attr, conv_w0, conv_b0.reshape(1, d0))

    part0 = _sc_edge_aggregate(h0, eidx_flat, n_pad)

    h1 = pl.pallas_call(
        _combine_linear_kernel,
        grid=(nblk,),
        in_specs=[
            pl.BlockSpec((_NC, blk, d0), lambda i: (0, i, 0)),
            pl.BlockSpec((d0, d1), lambda i: (0, 0)),
            pl.BlockSpec((1, d1), lambda i: (0, 0)),
        ],
        out_specs=pl.BlockSpec((blk, d1), lambda i: (i, 0)),
        out_shape=jax.ShapeDtypeStruct((n, d1), jnp.float32),
    )(part0, conv_w1, conv_b1.reshape(1, d1))

    part1 = _sc_edge_aggregate(h1, eidx_flat, n_pad)

    bat3 = batching.reshape(nblk, 1, blk)
    out = pl.pallas_call(
        functools.partial(_pool_fc_kernel, g=g, blk=blk, nblk=nblk),
        grid=(nblk,),
        in_specs=[
            pl.BlockSpec((_NC, blk, d1), lambda i: (0, i, 0)),
            pl.BlockSpec((1, 1, blk), lambda i: (i, 0, 0)),
            pl.BlockSpec((d1, f0), lambda i: (0, 0)),
            pl.BlockSpec((1, f0), lambda i: (0, 0)),
            pl.BlockSpec((f0, f1), lambda i: (0, 0)),
            pl.BlockSpec((1, f1), lambda i: (0, 0)),
        ],
        out_specs=pl.BlockSpec((g, f1), lambda i: (0, 0)),
        out_shape=jax.ShapeDtypeStruct((g, f1), jnp.float32),
        scratch_shapes=[pltpu.VMEM((g, d1), jnp.float32)],
    )(part1, bat3, fc_w0, fc_b0.reshape(1, f0), fc_w1, fc_b1.reshape(1, f1))

    return out


# speedup vs baseline: 1.2208x; 1.0038x over previous
import functools

import jax
import jax.numpy as jnp
from jax import lax
from jax.experimental import pallas as pl
from jax.experimental.pallas import tpu as pltpu
from jax.experimental.pallas import tpu_sc as plsc

_NC = 2
_NS = 16
_NW = _NC * _NS

_CH = 80

_NB = 4


def _sc_edge_aggregate(h, eidx_flat, n_pad):
    _, d = h.shape
    e = eidx_flat.shape[0] // 2
    per_w = e // _NW
    n_chunks = per_w // _CH
    n_groups = n_chunks // _NB
    n_tail = n_chunks - n_groups * _NB
    rows_per_tile = n_pad // _NS

    mesh = plsc.VectorSubcoreMesh(
        core_axis_name="c", subcore_axis_name="s",
        num_cores=_NC, num_subcores=_NS)

    @functools.partial(
        pl.kernel,
        out_type=jax.ShapeDtypeStruct((_NC, n_pad, d), jnp.float32),
        mesh=mesh,
        scratch_types=[
            pltpu.VMEM((2 * _NB * _CH,), jnp.int32),
            pltpu.VMEM((2, _NB, _CH), jnp.int32),
            pltpu.VMEM((_NB * _CH, d), jnp.float32),
            pltpu.VMEM_SHARED((n_pad, d), jnp.float32),
            pltpu.SemaphoreType.DMA,
            pltpu.SemaphoreType.DMA,
            pltpu.SemaphoreType.DMA,
            pltpu.SemaphoreType.DMA,
        ],
    )
    def edge_agg(h_hbm, eidx_hbm, out_hbm,
                 sidx, didx, rows, acc, sem_i0, sem_i1, sem_g, sem_s):
        c = lax.axis_index("c")
        s = lax.axis_index("s")
        wid = s * _NC + c
        base_w = wid * per_w

        pltpu.async_copy(
            eidx_hbm.at[pl.ds(base_w, _NB * _CH)],
            sidx.at[pl.ds(0, _NB * _CH)], sem_i0)
        for b in range(_NB):
            pltpu.async_copy(
                eidx_hbm.at[pl.ds(e + base_w + b * _CH, _CH)], didx.at[0, b],
                sem_i0)
        zv = jnp.zeros((16,), jnp.float32)

        def zrow(r, carry):
            for k8 in range(d // 16):
                rows[r, pl.ds(k8 * 16, 16)] = zv
            return carry

        lax.fori_loop(0, _CH, zrow, 0)
        z_full = rows_per_tile // _CH
        z_rem = rows_per_tile - z_full * _CH
        for zb in range(z_full):
            pltpu.sync_copy(
                rows.at[pl.ds(0, _CH)],
                acc.at[pl.ds(s * rows_per_tile + zb * _CH, _CH)])
        if z_rem:
            pltpu.sync_copy(
                rows.at[pl.ds(0, z_rem)],
                acc.at[pl.ds(s * rows_per_tile + z_full * _CH, z_rem)])
        plsc.subcore_barrier()

        def group(g, carry):
            p = lax.rem(g, 2)
            @pl.when(g > 0)
            def _():
                pltpu.make_async_copy(
                    h_hbm.at[pl.ds(0, _NB * _CH)], rows, sem_s).wait()
            @pl.when(g + 1 < n_groups)
            def _():
                base_n = base_w + (g + 1) * _NB * _CH

                @pl.when(p == 0)
                def _():
                    pltpu.async_copy(
                        eidx_hbm.at[pl.ds(base_n, _NB * _CH)],
                        sidx.at[pl.ds(_NB * _CH, _NB * _CH)], sem_i1)
                    for b in range(_NB):
                        pltpu.async_copy(
                            eidx_hbm.at[pl.ds(e + base_n + b * _CH, _CH)],
                            didx.at[1, b], sem_i1)

                @pl.when(p == 1)
                def _():
                    pltpu.async_copy(
                        eidx_hbm.at[pl.ds(base_n, _NB * _CH)],
                        sidx.at[pl.ds(0, _NB * _CH)], sem_i0)
                    for b in range(_NB):
                        pltpu.async_copy(
                            eidx_hbm.at[pl.ds(e + base_n + b * _CH, _CH)],
                            didx.at[0, b], sem_i0)

            @pl.when(p == 0)
            def _():
                pltpu.make_async_copy(
                    eidx_hbm.at[pl.ds(0, _NB * _CH)],
                    sidx.at[pl.ds(0, _NB * _CH)], sem_i0).wait()
                pltpu.make_async_copy(
                    eidx_hbm.at[pl.ds(0, _NB * _CH)],
                    sidx.at[pl.ds(0, _NB * _CH)], sem_i0).wait()

            @pl.when(p == 1)
            def _():
                pltpu.make_async_copy(
                    eidx_hbm.at[pl.ds(0, _NB * _CH)],
                    sidx.at[pl.ds(0, _NB * _CH)], sem_i1).wait()
                pltpu.make_async_copy(
                    eidx_hbm.at[pl.ds(0, _NB * _CH)],
                    sidx.at[pl.ds(0, _NB * _CH)], sem_i1).wait()

            gd = []
            for b in range(_NB):
                gd.append(pltpu.async_copy(
                    h_hbm.at[sidx.at[pl.ds(p * _NB * _CH + b * _CH, _CH)]],
                    rows.at[pl.ds(b * _CH, _CH)], sem_g))
            for b in range(_NB):
                gd[b].wait()
                pltpu.async_copy(
                    rows.at[pl.ds(b * _CH, _CH)], acc.at[didx.at[p, b]],
                    sem_s, add=True)
            return carry

        lax.fori_loop(0, n_groups, group, 0)
        pltpu.make_async_copy(
            h_hbm.at[pl.ds(0, _NB * _CH)], rows, sem_s).wait()

        for t in range(n_tail):
            base_t = base_w + (n_groups * _NB + t) * _CH
            pltpu.sync_copy(
                eidx_hbm.at[pl.ds(base_t, _CH)], sidx.at[pl.ds(0, _CH)])
            pltpu.sync_copy(eidx_hbm.at[pl.ds(e + base_t, _CH)], didx.at[0, 0])
            pltpu.async_copy(
                h_hbm.at[sidx.at[pl.ds(0, _CH)]],
                rows.at[pl.ds(0, _CH)], sem_g).wait()
            pltpu.async_copy(
                rows.at[pl.ds(0, _CH)], acc.at[didx.at[0, 0]], sem_s,
                add=True).wait()

        plsc.subcore_barrier()

        pltpu.sync_copy(
            acc.at[pl.ds(s * rows_per_tile, rows_per_tile)],
            out_hbm.at[c, pl.ds(s * rows_per_tile, rows_per_tile)])

    return edge_agg(h, eidx_flat)


def _dot_bf16x3(x, w):
    xh = x.astype(jnp.bfloat16)
    wh = w.astype(jnp.bfloat16)
    xl = (x - xh.astype(jnp.float32)).astype(jnp.bfloat16)
    wl = (w - wh.astype(jnp.float32)).astype(jnp.bfloat16)
    f = jnp.float32
    return (jnp.dot(xh, wh, preferred_element_type=f)
            + jnp.dot(xh, wl, preferred_element_type=f)
            + jnp.dot(xl, wh, preferred_element_type=f))


def _linear_kernel(x_ref, w_ref, b_ref, o_ref):
    o_ref[...] = _dot_bf16x3(x_ref[...], w_ref[...]) + b_ref[...]


def _combine_linear_kernel(p_ref, w_ref, b_ref, o_ref):
    h = jnp.maximum(p_ref[0] + p_ref[1], 0.0)
    o_ref[...] = _dot_bf16x3(h, w_ref[...]) + b_ref[...]


def _pool_fc_kernel(p_ref, bat_ref, fw0_ref, fb0_ref, fw1_ref, fb1_ref,
                    o_ref, pooled_acc, *, g, blk, nblk):
    i = pl.program_id(0)

    @pl.when(i == 0)
    def _():
        pooled_acc[...] = jnp.zeros_like(pooled_acc)

    h = jnp.maximum(p_ref[0] + p_ref[1], 0.0)
    b = bat_ref[0]
    seg = lax.broadcasted_iota(jnp.int32, (g, blk), 0)
    onehot = (seg == b).astype(jnp.bfloat16)
    hh = h.astype(jnp.bfloat16)
    hl = (h - hh.astype(jnp.float32)).astype(jnp.bfloat16)
    pooled_acc[...] += (
        jnp.dot(onehot, hh, preferred_element_type=jnp.float32)
        + jnp.dot(onehot, hl, preferred_element_type=jnp.float32))

    @pl.when(i == nblk - 1)
    def _():
        pooled = pooled_acc[...]
        z = jnp.dot(pooled, fw0_ref[...], preferred_element_type=jnp.float32,
                    precision=lax.Precision.HIGHEST) + fb0_ref[...]
        z = jnp.dot(z, fw1_ref[...], preferred_element_type=jnp.float32,
                    precision=lax.Precision.HIGHEST) + fb1_ref[...]
        z = z - jnp.max(z, axis=1, keepdims=True)
        ez = jnp.exp(z)
        o_ref[...] = ez / jnp.sum(ez, axis=1, keepdims=True)


def kernel(node_attr, edge_index, batching, conv_w0, conv_b0, conv_w1,
           conv_b1, fc_w0, fc_b0, fc_w1, fc_b1):
    n, d_in = node_attr.shape
    d0 = conv_w0.shape[1]
    d1 = conv_w1.shape[1]
    f0 = fc_w0.shape[1]
    f1 = fc_w1.shape[1]
    g = 64

    eidx_flat = edge_index.reshape(-1)

    blk = 1000
    nblk = n // blk
    n_pad = ((n + 8 * _NS - 1) // (8 * _NS)) * (8 * _NS)

    h0 = pl.pallas_call(
        _linear_kernel,
        grid=(nblk,),
        in_specs=[
            pl.BlockSpec((blk, d_in), lambda i: (i, 0)),
            pl.BlockSpec((d_in, d0), lambda i: (0, 0)),
            pl.BlockSpec((1, d0), lambda i: (0, 0)),
        ],
        out_specs=pl.BlockSpec((blk, d0), lambda i: (i, 0)),
        out_shape=jax.ShapeDtypeStruct((n, d0), jnp.float32),
    )(node_attr, conv_w0, conv_b0.reshape(1, d0))

    part0 = _sc_edge_aggregate(h0, eidx_flat, n_pad)

    h1 = pl.pallas_call(
        _combine_linear_kernel,
        grid=(nblk,),
        in_specs=[
            pl.BlockSpec((_NC, blk, d0), lambda i: (0, i, 0)),
            pl.BlockSpec((d0, d1), lambda i: (0, 0)),
            pl.BlockSpec((1, d1), lambda i: (0, 0)),
        ],
        out_specs=pl.BlockSpec((blk, d1), lambda i: (i, 0)),
        out_shape=jax.ShapeDtypeStruct((n, d1), jnp.float32),
    )(part0, conv_w1, conv_b1.reshape(1, d1))

    part1 = _sc_edge_aggregate(h1, eidx_flat, n_pad)

    bat3 = batching.reshape(nblk, 1, blk)
    out = pl.pallas_call(
        functools.partial(_pool_fc_kernel, g=g, blk=blk, nblk=nblk),
        grid=(nblk,),
        in_specs=[
            pl.BlockSpec((_NC, blk, d1), lambda i: (0, i, 0)),
            pl.BlockSpec((1, 1, blk), lambda i: (i, 0, 0)),
            pl.BlockSpec((d1, f0), lambda i: (0, 0)),
            pl.BlockSpec((1, f0), lambda i: (0, 0)),
            pl.BlockSpec((f0, f1), lambda i: (0, 0)),
            pl.BlockSpec((1, f1), lambda i: (0, 0)),
        ],
        out_specs=pl.BlockSpec((g, f1), lambda i: (0, 0)),
        out_shape=jax.ShapeDtypeStruct((g, f1), jnp.float32),
        scratch_shapes=[pltpu.VMEM((g, d1), jnp.float32)],
    )(part1, bat3, fc_w0, fc_b0.reshape(1, f0), fc_w1, fc_b1.reshape(1, f1))

    return out
